# Initial kernel scaffold; baseline (speedup 1.0000x reference)
import jax
import jax.numpy as jnp
from jax.experimental import pallas as pl


def kernel(first_emb, second_emb, context_emb, v_i, v_j):
    raise NotImplementedError("write your pallas kernel here")


# speedup vs baseline: 1.7470x; 1.7470x over previous
import functools

import jax
import jax.numpy as jnp
from jax import lax
from jax.experimental import pallas as pl
from jax.experimental.pallas import tpu as pltpu
from jax.experimental.pallas import tpu_sc as plsc

D = 128
B = 16384
NC, NS = 2, 16
NW = NC * NS
ROWS_PER_W = 2 * B // NW
CHUNK = 128
NCHUNK = ROWS_PER_W // CHUNK
NBUF = 4


def _line_body(table_hbm, idx_hbm, out_hbm, idx_v, rows_v, s0, s1, s2, s3):
    sems = (s0, s1, s2, s3)
    wid = lax.axis_index("s") * NC + lax.axis_index("c")
    base = wid * ROWS_PER_W

    pltpu.sync_copy(idx_hbm.at[wid], idx_v)

    def gather(c):
        b = c % NBUF
        return pltpu.async_copy(table_hbm.at[idx_v.at[c]], rows_v.at[b], sems[b])

    def put(c):
        b = c % NBUF
        return pltpu.async_copy(
            rows_v.at[b], out_hbm.at[pl.ds(base + c * CHUNK, CHUNK)], sems[b]
        )

    g = {}
    p = {}
    for c in range(NBUF):
        g[c] = gather(c)
    for c in range(NCHUNK):
        if c >= 1 and c + NBUF - 1 < NCHUNK:
            p[c - 1].wait()
            g[c + NBUF - 1] = gather(c + NBUF - 1)
        g[c].wait()
        p[c] = put(c)
    for c in range(NCHUNK - NBUF, NCHUNK):
        p[c].wait()


@functools.partial(jax.jit, static_argnames=())
def _line_gather(first_emb, idx):
    mesh = plsc.VectorSubcoreMesh(
        core_axis_name="c", subcore_axis_name="s", num_cores=NC, num_subcores=NS
    )
    f = functools.partial(
        pl.kernel,
        out_type=jax.ShapeDtypeStruct((2 * B, D), jnp.float32),
        mesh=mesh,
        scratch_types=[
            pltpu.VMEM((NCHUNK, CHUNK), jnp.int32),
            pltpu.VMEM((NBUF, CHUNK, D), jnp.float32),
            pltpu.SemaphoreType.DMA,
            pltpu.SemaphoreType.DMA,
            pltpu.SemaphoreType.DMA,
            pltpu.SemaphoreType.DMA,
        ],
    )(_line_body)
    return f(first_emb, idx)


def kernel(first_emb, second_emb, context_emb, v_i, v_j):
    idx = jnp.stack((v_i, v_j)).reshape(NW, NCHUNK, CHUNK)
    out = _line_gather(first_emb, idx)
    return out.reshape(2, B, D)


# speedup vs baseline: 1.8141x; 1.0384x over previous
import functools

import jax
import jax.numpy as jnp
from jax import lax
from jax.experimental import pallas as pl
from jax.experimental.pallas import tpu as pltpu
from jax.experimental.pallas import tpu_sc as plsc

D = 128
B = 16384
NC, NS = 2, 16
NW = NC * NS
ROWS_PER_W = B // (NW // 2)
CHUNK = 128
NCHUNK = ROWS_PER_W // CHUNK
NBUF = 7


def _line_body(table_hbm, vi_hbm, vj_hbm, out_hbm, idx_v, rows_v, *sems):
    wid = lax.axis_index("s") * NC + lax.axis_index("c")
    pos = lax.rem(wid, 16)

    def pipeline(idx_blk, out_half):
        base = pos * ROWS_PER_W
        pltpu.sync_copy(idx_blk, idx_v)

        def gather(c, b):
            return pltpu.async_copy(
                table_hbm.at[idx_v.at[c]], rows_v.at[b], sems[b]
            )

        def put(c, b):
            return pltpu.async_copy(
                rows_v.at[b], out_half.at[pl.ds(base + c * CHUNK, CHUNK)], sems[b]
            )

        g = {}
        p = {}
        for c in range(NBUF):
            g[c] = gather(c, c)
        for c in range(NCHUNK):
            if c == 2 and NCHUNK > NBUF:
                p[0].wait()
                g[NBUF] = gather(NBUF, 0)
            g[c].wait()
            p[c] = put(c, c % NBUF)
        for c in range(1, NCHUNK):
            p[c].wait()

    @pl.when(wid < 16)
    def _():
        pipeline(vi_hbm.at[pos], out_hbm.at[0])

    @pl.when(wid >= 16)
    def _():
        pipeline(vj_hbm.at[pos], out_hbm.at[1])


@jax.jit
def _line_gather(first_emb, vi, vj):
    mesh = plsc.VectorSubcoreMesh(
        core_axis_name="c", subcore_axis_name="s", num_cores=NC, num_subcores=NS
    )
    f = functools.partial(
        pl.kernel,
        out_type=jax.ShapeDtypeStruct((2, B, D), jnp.float32),
        mesh=mesh,
        scratch_types=[
            pltpu.VMEM((NCHUNK, CHUNK), jnp.int32),
            pltpu.VMEM((NBUF, CHUNK, D), jnp.float32),
        ]
        + [pltpu.SemaphoreType.DMA] * NBUF,
    )(_line_body)
    return f(first_emb, vi, vj)


def kernel(first_emb, second_emb, context_emb, v_i, v_j):
    vi = v_i.reshape(NW // 2, NCHUNK, CHUNK)
    vj = v_j.reshape(NW // 2, NCHUNK, CHUNK)
    return _line_gather(first_emb, vi, vj)
